# baseline (speedup 1.0000x reference)
import functools

import jax
import jax.numpy as jnp
from jax import lax
from jax.experimental import pallas as pl
from jax.experimental.pallas import tpu as pltpu
from jax.experimental.pallas import tpu_sc as plsc

F = 32
FH = F // 2
NSUB = 16
NW = 32
NBUF = 4
SEC = 50
DEG_GRP = 5
_K_DEG = {51200: 512, 12800: 512, 3200: 320, 800: 160, 200: 40, 17000: 425,
          68200: 440}
_K_SCAT = {102400: 400, 25600: 512, 6400: 256, 1600: 320, 400: 100,
           34000: 425}


def _pad_n(n):
    return ((n + 127) // 128) * 128


@functools.cache
def _deg_kernel(n_pad, e):
    per_w = e // NW
    K = _K_DEG[per_w]
    nch = per_w // K
    rz = n_pad // NSUB
    mesh = plsc.VectorSubcoreMesh(core_axis_name="c", subcore_axis_name="s")

    K_pad = ((K + 15) // 16) * 16

    def body(dstr, zer, out, dst_v, ones_v, sem, acc):
        c = lax.axis_index("c")
        s = lax.axis_index("s")
        w = c * NSUB + s
        pltpu.sync_copy(zer, acc.at[pl.ds(s * rz, rz)])
        pltpu.sync_copy(dstr.at[w], dst_v)
        for i in range(K_pad // 16):
            ones_v[pl.ds(i * 16, 16)] = jnp.ones((16,), jnp.float32)
        ones_s = ones_v if K == K_pad else ones_v.at[pl.ds(0, K)]
        plsc.subcore_barrier()

        def grp(g, carry):
            base = g * DEG_GRP
            for t in range(DEG_GRP):
                pltpu.async_copy(ones_s, acc.at[dst_v.at[base + t]], sem,
                                 add=True)
            for _ in range(DEG_GRP):
                pltpu.make_async_copy(ones_s, acc.at[dst_v.at[0]], sem).wait()
            return carry

        lax.fori_loop(0, nch // DEG_GRP, grp, 0)
        plsc.subcore_barrier()
        pltpu.sync_copy(acc.at[pl.ds(s * rz, rz)],
                        out.at[pl.ds(c * n_pad + s * rz, rz)])

    return pl.kernel(
        body,
        out_type=jax.ShapeDtypeStruct((2 * n_pad,), jnp.float32),
        mesh=mesh,
        compiler_params=pltpu.CompilerParams(use_tc_tiling_on_sc=False),
        scratch_types=[
            pltpu.VMEM((nch, K), jnp.int32),
            pltpu.VMEM((K_pad,), jnp.float32),
            pltpu.SemaphoreType.DMA,
            pltpu.VMEM_SHARED((n_pad,), jnp.float32),
        ],
    )


@functools.cache
def _scat_kernel(n_pad, e):
    per_s = e // NSUB
    K = _K_SCAT[per_s]
    nch = per_s // K
    sec = next(d for d in range(min(nch, SEC), 0, -1) if nch % d == 0)
    nsec = nch // sec
    assert sec >= NBUF
    rz = n_pad // NSUB
    mesh = plsc.VectorSubcoreMesh(core_axis_name="c", subcore_axis_name="s")

    def body(hps, srcr, dstr, zer, out, src_v, dst_v, bufs, gsem, acc):
        c = lax.axis_index("c")
        s = lax.axis_index("s")
        hp_half = hps.at[c]
        pltpu.sync_copy(zer, acc.at[pl.ds(s * rz, rz)])
        plsc.subcore_barrier()
        for t in range(nsec):
            pltpu.sync_copy(srcr.at[s].at[pl.ds(t * sec, sec)], src_v)
            pltpu.sync_copy(dstr.at[s].at[pl.ds(t * sec, sec)], dst_v)
            for b in range(NBUF):
                pltpu.async_copy(hp_half.at[src_v.at[b]], bufs.at[b],
                                 gsem.at[b])

            def step(i, carry):
                p = lax.rem(i, NBUF)
                pltpu.make_async_copy(hp_half.at[src_v.at[0]], bufs.at[p],
                                      gsem.at[p]).wait()
                pltpu.sync_copy(bufs.at[p], acc.at[dst_v.at[i]], add=True)
                nxt = i + NBUF

                @pl.when(nxt < sec)
                def _():
                    pltpu.async_copy(hp_half.at[src_v.at[nxt]], bufs.at[p],
                                     gsem.at[p])

                return carry

            lax.fori_loop(0, sec, step, 0)
        plsc.subcore_barrier()
        pltpu.sync_copy(acc.at[pl.ds(s * rz, rz)],
                        out.at[c].at[pl.ds(s * rz, rz)])

    return pl.kernel(
        body,
        out_type=jax.ShapeDtypeStruct((2, n_pad, FH), jnp.float32),
        mesh=mesh,
        compiler_params=pltpu.CompilerParams(use_tc_tiling_on_sc=False),
        scratch_types=[
            pltpu.VMEM((sec, K), jnp.int32),
            pltpu.VMEM((sec, K), jnp.int32),
            pltpu.VMEM((NBUF, K, FH), jnp.float32),
            pltpu.SemaphoreType.DMA((NBUF,)),
            pltpu.VMEM_SHARED((n_pad, FH), jnp.float32),
        ],
    )


def _tc_block(n_pad):
    b = 6400 if n_pad % 6400 == 0 else n_pad
    return b, n_pad // b


def _split(h):
    return jnp.stack([h[:, :FH], h[:, FH:]], axis=0)


@functools.cache
def _prep_kernel(n_pad, din, interpret=False):
    B, g = _tc_block(n_pad)

    def body(x, d0, d1, w1, dinv_o, hps_o):
        deg = d0[...] + d1[...] + 1.0
        dinv = lax.rsqrt(deg)
        h = jnp.dot(x[...], w1[...], preferred_element_type=jnp.float32)
        dinv_o[...] = dinv
        hps_o[...] = _split(h * dinv)

    return pl.pallas_call(
        body,
        grid=(g,),
        in_specs=[
            pl.BlockSpec((B, din), lambda i: (i, 0)),
            pl.BlockSpec((B, 1), lambda i: (i, 0)),
            pl.BlockSpec((B, 1), lambda i: (i, 0)),
            pl.BlockSpec((din, F), lambda i: (0, 0)),
        ],
        out_specs=[
            pl.BlockSpec((B, 1), lambda i: (i, 0)),
            pl.BlockSpec((2, B, FH), lambda i: (0, i, 0)),
        ],
        out_shape=[
            jax.ShapeDtypeStruct((n_pad, 1), jnp.float32),
            jax.ShapeDtypeStruct((2, n_pad, FH), jnp.float32),
        ],
        interpret=interpret,
    )


@functools.cache
def _mid_kernel(n_pad, interpret=False):
    B, g = _tc_block(n_pad)

    def body(parts, hps, dinv, bvec, w, hpn_o):
        pr = parts[...]
        hr = hps[...]
        agg = jnp.concatenate([pr[0], pr[1]], axis=1)
        hp = jnp.concatenate([hr[0], hr[1]], axis=1)
        z = dinv[...] * (agg + hp) + bvec[...]
        xn = jnp.maximum(z, 0.0)
        hpn = jnp.dot(xn, w[...], preferred_element_type=jnp.float32) * dinv[...]
        hpn_o[...] = _split(hpn)

    return pl.pallas_call(
        body,
        grid=(g,),
        in_specs=[
            pl.BlockSpec((2, B, FH), lambda i: (0, i, 0)),
            pl.BlockSpec((2, B, FH), lambda i: (0, i, 0)),
            pl.BlockSpec((B, 1), lambda i: (i, 0)),
            pl.BlockSpec((1, F), lambda i: (0, 0)),
            pl.BlockSpec((F, F), lambda i: (0, 0)),
        ],
        out_specs=pl.BlockSpec((2, B, FH), lambda i: (0, i, 0)),
        out_shape=jax.ShapeDtypeStruct((2, n_pad, FH), jnp.float32),
        interpret=interpret,
    )


@functools.cache
def _fin_kernel(n_pad, interpret=False):
    B, g = _tc_block(n_pad)

    def body(parts, hps, dinv, bvec, h_o):
        pr = parts[...]
        hr = hps[...]
        agg = jnp.concatenate([pr[0], pr[1]], axis=1)
        hp = jnp.concatenate([hr[0], hr[1]], axis=1)
        h_o[...] = dinv[...] * (agg + hp) + bvec[...]

    return pl.pallas_call(
        body,
        grid=(g,),
        in_specs=[
            pl.BlockSpec((2, B, FH), lambda i: (0, i, 0)),
            pl.BlockSpec((2, B, FH), lambda i: (0, i, 0)),
            pl.BlockSpec((B, 1), lambda i: (i, 0)),
            pl.BlockSpec((1, F), lambda i: (0, 0)),
        ],
        out_specs=pl.BlockSpec((B, F), lambda i: (i, 0)),
        out_shape=jax.ShapeDtypeStruct((n_pad, F), jnp.float32),
        interpret=interpret,
    )


@functools.cache
def _fuse_kernel(n1, interpret=False):
    B = 2048
    g = n1 // B

    def rep(x, k):
        r = x.shape[0]
        return jnp.broadcast_to(x[:, None, :], (r, k, F)).reshape(r * k, F)

    def body(h1, h2, h3, h4, h5, fw, fb, o):
        sm = (h1[...] + rep(h2[...], 4) + rep(h3[...], 16)
              + rep(h4[...], 64) + rep(h5[...], 256))
        sm = jnp.maximum(sm, 0.0)
        o[...] = jnp.dot(sm, fw[...],
                         preferred_element_type=jnp.float32) + fb[...]

    return pl.pallas_call(
        body,
        grid=(g,),
        in_specs=[
            pl.BlockSpec((B, F), lambda i: (i, 0)),
            pl.BlockSpec((B // 4, F), lambda i: (i, 0)),
            pl.BlockSpec((B // 16, F), lambda i: (i, 0)),
            pl.BlockSpec((B // 64, F), lambda i: (i, 0)),
            pl.BlockSpec((B // 256, F), lambda i: (i, 0)),
            pl.BlockSpec((F, 1), lambda i: (0, 0)),
            pl.BlockSpec((1, 1), lambda i: (0, 0)),
        ],
        out_specs=pl.BlockSpec((B, 1), lambda i: (i, 0)),
        out_shape=jax.ShapeDtypeStruct((n1, 1), jnp.float32),
        interpret=interpret,
    )


def _group(xs, es, Ws_l, bs_l, degs):
    n_pads = [_pad_n(x.shape[0]) for x in xs]
    bases = [sum(n_pads[:i]) for i in range(len(n_pads))]
    ncat = sum(n_pads)
    ecat = sum(e.shape[1] for e in es)
    ks = _K_SCAT[ecat // NSUB]
    if len(xs) == 1:
        src, dst = es[0][0], es[0][1]
    else:
        src = jnp.concatenate([e[0] + b for e, b in zip(es, bases)])
        dst = jnp.concatenate([e[1] + b for e, b in zip(es, bases)])
    src_s = src.reshape(NSUB, (ecat // NSUB) // ks, ks)
    dst_s = dst.reshape(NSUB, (ecat // NSUB) // ks, ks)
    rz = ncat // NSUB
    zerf = jnp.zeros((rz, FH), jnp.float32)

    dinvs, hps_l = [], []
    for x, npad, Ws, (d0, d1) in zip(xs, n_pads, Ws_l, degs):
        n = x.shape[0]
        xp = x if npad == n else jnp.pad(x, ((0, npad - n), (0, 0)))
        dinv, hps = _prep_kernel(npad, x.shape[1])(xp, d0, d1, Ws[0])
        dinvs.append(dinv)
        hps_l.append(hps)
    hs = [None] * len(xs)
    for j in range(3):
        hp_cat = hps_l[0] if len(xs) == 1 else jnp.concatenate(hps_l, axis=1)
        parts = _scat_kernel(ncat, ecat)(hp_cat, src_s, dst_s, zerf)
        new_hps = []
        for li, (npad, b, Ws, bs) in enumerate(zip(n_pads, bases, Ws_l, bs_l)):
            pl_ = lax.slice(parts, (0, b, 0), (2, b + npad, FH))
            bvec = bs[j].reshape(1, F)
            if j < 2:
                new_hps.append(_mid_kernel(npad)(pl_, hps_l[li], dinvs[li],
                                                 bvec, Ws[j + 1]))
            else:
                hs[li] = _fin_kernel(npad)(pl_, hps_l[li], dinvs[li], bvec)
        hps_l = new_hps
    return hs


@jax.jit
def _forward(x1, x2, x3, x4, x5, e1, e2, e3, e4, e5, params):
    xs = (x1, x2, x3, x4, x5)
    es = (e1, e2, e3, e4, e5)
    Ws_all = [tuple(params['conv%d%d_W' % (i, j)] for j in (1, 2, 3))
              for i in range(1, 6)]
    bs_all = [tuple(params['conv%d%d_b' % (i, j)] for j in (1, 2, 3))
              for i in range(1, 6)]

    n_pads = [_pad_n(x.shape[0]) for x in xs]
    bases = [sum(n_pads[:i]) for i in range(len(n_pads))]
    ncat = sum(n_pads)
    ecat = sum(e.shape[1] for e in es)
    kd = _K_DEG[ecat // NW]
    dst_all = jnp.concatenate([e[1] + b for e, b in zip(es, bases)])
    dst_d = dst_all.reshape(NW, (ecat // NW) // kd, kd)
    degp = _deg_kernel(ncat, ecat)(dst_d,
                                   jnp.zeros((ncat // NSUB,), jnp.float32))
    degs = []
    for npad, b in zip(n_pads, bases):
        d0 = lax.slice(degp, (b,), (b + npad,)).reshape(npad, 1)
        d1 = lax.slice(degp, (ncat + b,), (ncat + b + npad,)).reshape(npad, 1)
        degs.append((d0, d1))

    h1 = _group(xs[:1], es[:1], Ws_all[:1], bs_all[:1], degs[:1])[0]
    h2 = _group(xs[1:2], es[1:2], Ws_all[1:2], bs_all[1:2], degs[1:2])[0]
    h3 = _group(xs[2:3], es[2:3], Ws_all[2:3], bs_all[2:3], degs[2:3])[0]
    h4 = _group(xs[3:4], es[3:4], Ws_all[3:4], bs_all[3:4], degs[3:4])[0]
    h5 = _group(xs[4:5], es[4:5], Ws_all[4:5], bs_all[4:5], degs[4:5])[0]
    n1 = x1.shape[0]
    return _fuse_kernel(n1)(h1, h2, h3, h4, h5,
                            params['fc_W'].reshape(F, 1),
                            params['fc_b'].reshape(1, 1))


def kernel(x1, x2, x3, x4, x5, edge_index1, edge_index2, edge_index3,
           edge_index4, edge_index5, params):
    return _forward(x1, x2, x3, x4, x5, edge_index1, edge_index2,
                    edge_index3, edge_index4, edge_index5, params)


# speedup vs baseline: 1.0066x; 1.0066x over previous
import functools

import jax
import jax.numpy as jnp
from jax import lax
from jax.experimental import pallas as pl
from jax.experimental.pallas import tpu as pltpu
from jax.experimental.pallas import tpu_sc as plsc

F = 32
FH = F // 2
NSUB = 16
NW = 32
NBUF = 4
SEC = 50
DEG_GRP = 5
_K_DEG = {51200: 512, 12800: 512, 3200: 320, 800: 160, 200: 40, 17000: 425,
          68200: 440}
_K_SCAT = {102400: 400, 25600: 512, 6400: 256, 1600: 320, 400: 100,
           34000: 425}


def _pad_n(n):
    return ((n + 127) // 128) * 128


@functools.cache
def _deg_kernel(n_pad, e):
    per_w = e // NW
    K = _K_DEG[per_w]
    nch = per_w // K
    rz = n_pad // NSUB
    mesh = plsc.VectorSubcoreMesh(core_axis_name="c", subcore_axis_name="s")

    K_pad = ((K + 15) // 16) * 16

    def body(dstr, zer, out, dst_v, ones_v, sem, acc):
        c = lax.axis_index("c")
        s = lax.axis_index("s")
        w = c * NSUB + s
        pltpu.sync_copy(zer, acc.at[pl.ds(s * rz, rz)])
        pltpu.sync_copy(dstr.at[w], dst_v)
        for i in range(K_pad // 16):
            ones_v[pl.ds(i * 16, 16)] = jnp.ones((16,), jnp.float32)
        ones_s = ones_v if K == K_pad else ones_v.at[pl.ds(0, K)]
        plsc.subcore_barrier()

        def grp(g, carry):
            base = g * DEG_GRP
            for t in range(DEG_GRP):
                pltpu.async_copy(ones_s, acc.at[dst_v.at[base + t]], sem,
                                 add=True)
            for _ in range(DEG_GRP):
                pltpu.make_async_copy(ones_s, acc.at[dst_v.at[0]], sem).wait()
            return carry

        lax.fori_loop(0, nch // DEG_GRP, grp, 0)
        plsc.subcore_barrier()
        pltpu.sync_copy(acc.at[pl.ds(s * rz, rz)],
                        out.at[pl.ds(c * n_pad + s * rz, rz)])

    return pl.kernel(
        body,
        out_type=jax.ShapeDtypeStruct((2 * n_pad,), jnp.float32),
        mesh=mesh,
        compiler_params=pltpu.CompilerParams(use_tc_tiling_on_sc=False),
        scratch_types=[
            pltpu.VMEM((nch, K), jnp.int32),
            pltpu.VMEM((K_pad,), jnp.float32),
            pltpu.SemaphoreType.DMA,
            pltpu.VMEM_SHARED((n_pad,), jnp.float32),
        ],
    )


@functools.cache
def _scat_kernel(n_pad, e):
    per_s = e // NSUB
    K = _K_SCAT[per_s]
    nch = per_s // K
    sec = next(d for d in range(min(nch, SEC), 0, -1) if nch % d == 0)
    nsec = nch // sec
    assert sec >= NBUF
    rz = n_pad // NSUB
    mesh = plsc.VectorSubcoreMesh(core_axis_name="c", subcore_axis_name="s")

    def body(hps, srcr, dstr, zer, out, src_v, dst_v, bufs, gsem, acc):
        c = lax.axis_index("c")
        s = lax.axis_index("s")
        hp_half = hps.at[c]
        pltpu.sync_copy(zer, acc.at[pl.ds(s * rz, rz)])
        plsc.subcore_barrier()
        for t in range(nsec):
            pltpu.sync_copy(srcr.at[s].at[pl.ds(t * sec, sec)], src_v)
            pltpu.sync_copy(dstr.at[s].at[pl.ds(t * sec, sec)], dst_v)
            for b in range(NBUF):
                pltpu.async_copy(hp_half.at[src_v.at[b]], bufs.at[b],
                                 gsem.at[b])

            def step(i, carry):
                p = lax.rem(i, NBUF)
                pltpu.make_async_copy(hp_half.at[src_v.at[0]], bufs.at[p],
                                      gsem.at[p]).wait()
                pltpu.sync_copy(bufs.at[p], acc.at[dst_v.at[i]], add=True)
                nxt = i + NBUF

                @pl.when(nxt < sec)
                def _():
                    pltpu.async_copy(hp_half.at[src_v.at[nxt]], bufs.at[p],
                                     gsem.at[p])

                return carry

            lax.fori_loop(0, sec, step, 0)
        plsc.subcore_barrier()
        pltpu.sync_copy(acc.at[pl.ds(s * rz, rz)],
                        out.at[c].at[pl.ds(s * rz, rz)])

    return pl.kernel(
        body,
        out_type=jax.ShapeDtypeStruct((2, n_pad, FH), jnp.float32),
        mesh=mesh,
        compiler_params=pltpu.CompilerParams(use_tc_tiling_on_sc=False),
        scratch_types=[
            pltpu.VMEM((sec, K), jnp.int32),
            pltpu.VMEM((sec, K), jnp.int32),
            pltpu.VMEM((NBUF, K, FH), jnp.float32),
            pltpu.SemaphoreType.DMA((NBUF,)),
            pltpu.VMEM_SHARED((n_pad, FH), jnp.float32),
        ],
    )


def _tc_block(n_pad):
    b = 6400 if n_pad % 6400 == 0 else n_pad
    return b, n_pad // b


def _split(h):
    return jnp.stack([h[:, :FH], h[:, FH:]], axis=0)


@functools.cache
def _prep_kernel(n_pad, din, interpret=False):
    B, g = _tc_block(n_pad)

    def body(x, d0, d1, w1, dinv_o, hps_o):
        deg = d0[...] + d1[...] + 1.0
        dinv = lax.rsqrt(deg)
        h = jnp.dot(x[...], w1[...], preferred_element_type=jnp.float32)
        dinv_o[...] = dinv
        hps_o[...] = _split(h * dinv)

    return pl.pallas_call(
        body,
        grid=(g,),
        in_specs=[
            pl.BlockSpec((B, din), lambda i: (i, 0)),
            pl.BlockSpec((B, 1), lambda i: (i, 0)),
            pl.BlockSpec((B, 1), lambda i: (i, 0)),
            pl.BlockSpec((din, F), lambda i: (0, 0)),
        ],
        out_specs=[
            pl.BlockSpec((B, 1), lambda i: (i, 0)),
            pl.BlockSpec((2, B, FH), lambda i: (0, i, 0)),
        ],
        out_shape=[
            jax.ShapeDtypeStruct((n_pad, 1), jnp.float32),
            jax.ShapeDtypeStruct((2, n_pad, FH), jnp.float32),
        ],
        interpret=interpret,
    )


@functools.cache
def _mid_kernel(n_pad, interpret=False):
    B, g = _tc_block(n_pad)

    def body(parts, hps, dinv, bvec, w, hpn_o):
        pr = parts[...]
        hr = hps[...]
        agg = jnp.concatenate([pr[0], pr[1]], axis=1)
        hp = jnp.concatenate([hr[0], hr[1]], axis=1)
        z = dinv[...] * (agg + hp) + bvec[...]
        xn = jnp.maximum(z, 0.0)
        hpn = jnp.dot(xn, w[...], preferred_element_type=jnp.float32) * dinv[...]
        hpn_o[...] = _split(hpn)

    return pl.pallas_call(
        body,
        grid=(g,),
        in_specs=[
            pl.BlockSpec((2, B, FH), lambda i: (0, i, 0)),
            pl.BlockSpec((2, B, FH), lambda i: (0, i, 0)),
            pl.BlockSpec((B, 1), lambda i: (i, 0)),
            pl.BlockSpec((1, F), lambda i: (0, 0)),
            pl.BlockSpec((F, F), lambda i: (0, 0)),
        ],
        out_specs=pl.BlockSpec((2, B, FH), lambda i: (0, i, 0)),
        out_shape=jax.ShapeDtypeStruct((2, n_pad, FH), jnp.float32),
        interpret=interpret,
    )


@functools.cache
def _fin_kernel(n_pad, interpret=False):
    B, g = _tc_block(n_pad)

    def body(parts, hps, dinv, bvec, h_o):
        pr = parts[...]
        hr = hps[...]
        agg = jnp.concatenate([pr[0], pr[1]], axis=1)
        hp = jnp.concatenate([hr[0], hr[1]], axis=1)
        h_o[...] = dinv[...] * (agg + hp) + bvec[...]

    return pl.pallas_call(
        body,
        grid=(g,),
        in_specs=[
            pl.BlockSpec((2, B, FH), lambda i: (0, i, 0)),
            pl.BlockSpec((2, B, FH), lambda i: (0, i, 0)),
            pl.BlockSpec((B, 1), lambda i: (i, 0)),
            pl.BlockSpec((1, F), lambda i: (0, 0)),
        ],
        out_specs=pl.BlockSpec((B, F), lambda i: (i, 0)),
        out_shape=jax.ShapeDtypeStruct((n_pad, F), jnp.float32),
        interpret=interpret,
    )


@functools.cache
def _fuse_kernel(n1, interpret=False):
    B = 2048
    g = n1 // B

    def rep(x, k):
        r = x.shape[0]
        return jnp.broadcast_to(x[:, None, :], (r, k, F)).reshape(r * k, F)

    def body(h1, h2, h3, h4, h5, fw, fb, o):
        sm = (h1[...] + rep(h2[...], 4) + rep(h3[...], 16)
              + rep(h4[...], 64) + rep(h5[...], 256))
        sm = jnp.maximum(sm, 0.0)
        o[...] = jnp.dot(sm, fw[...],
                         preferred_element_type=jnp.float32) + fb[...]

    return pl.pallas_call(
        body,
        grid=(g,),
        in_specs=[
            pl.BlockSpec((B, F), lambda i: (i, 0)),
            pl.BlockSpec((B // 4, F), lambda i: (i, 0)),
            pl.BlockSpec((B // 16, F), lambda i: (i, 0)),
            pl.BlockSpec((B // 64, F), lambda i: (i, 0)),
            pl.BlockSpec((B // 256, F), lambda i: (i, 0)),
            pl.BlockSpec((F, 1), lambda i: (0, 0)),
            pl.BlockSpec((1, 1), lambda i: (0, 0)),
        ],
        out_specs=pl.BlockSpec((B, 1), lambda i: (i, 0)),
        out_shape=jax.ShapeDtypeStruct((n1, 1), jnp.float32),
        interpret=interpret,
    )


@jax.jit
def _forward(x1, x2, x3, x4, x5, e1, e2, e3, e4, e5, params):
    xs = (x1, x2, x3, x4, x5)
    es = (e1, e2, e3, e4, e5)
    nl = len(xs)
    Ws_all = [tuple(params['conv%d%d_W' % (i, j)] for j in (1, 2, 3))
              for i in range(1, 6)]
    bs_all = [tuple(params['conv%d%d_b' % (i, j)] for j in (1, 2, 3))
              for i in range(1, 6)]
    n_pads = [_pad_n(x.shape[0]) for x in xs]
    eNs = [e.shape[1] for e in es]

    degps = []
    for e, npad, eN in zip(es, n_pads, eNs):
        kd = _K_DEG[eN // NW]
        dst_d = e[1].reshape(NW, (eN // NW) // kd, kd)
        degps.append(_deg_kernel(npad, eN)(
            dst_d, jnp.zeros((npad // NSUB,), jnp.float32)))

    dinvs, hps_l = [], []
    for x, npad, Ws, degp in zip(xs, n_pads, Ws_all, degps):
        n = x.shape[0]
        xp = x if npad == n else jnp.pad(x, ((0, npad - n), (0, 0)))
        d0 = degp[:npad].reshape(npad, 1)
        d1 = degp[npad:].reshape(npad, 1)
        dinv, hps = _prep_kernel(npad, x.shape[1])(xp, d0, d1, Ws[0])
        dinvs.append(dinv)
        hps_l.append(hps)

    hs = [None] * nl
    for j in range(3):
        parts_l = []
        for e, npad, eN, hps in zip(es, n_pads, eNs, hps_l):
            ks = _K_SCAT[eN // NSUB]
            src_s = e[0].reshape(NSUB, (eN // NSUB) // ks, ks)
            dst_s = e[1].reshape(NSUB, (eN // NSUB) // ks, ks)
            zerf = jnp.zeros((npad // NSUB, FH), jnp.float32)
            parts_l.append(_scat_kernel(npad, eN)(hps, src_s, dst_s, zerf))
        new_hps = []
        for li, (npad, Ws, bs) in enumerate(zip(n_pads, Ws_all, bs_all)):
            bvec = bs[j].reshape(1, F)
            if j < 2:
                new_hps.append(_mid_kernel(npad)(parts_l[li], hps_l[li],
                                                 dinvs[li], bvec, Ws[j + 1]))
            else:
                hs[li] = _fin_kernel(npad)(parts_l[li], hps_l[li],
                                           dinvs[li], bvec)
        hps_l = new_hps

    n1 = x1.shape[0]
    return _fuse_kernel(n1)(hs[0], hs[1], hs[2], hs[3], hs[4],
                            params['fc_W'].reshape(F, 1),
                            params['fc_b'].reshape(1, 1))


def kernel(x1, x2, x3, x4, x5, edge_index1, edge_index2, edge_index3,
           edge_index4, edge_index5, params):
    return _forward(x1, x2, x3, x4, x5, edge_index1, edge_index2,
                    edge_index3, edge_index4, edge_index5, params)


# speedup vs baseline: 1.0193x; 1.0126x over previous
import functools

import jax
import jax.numpy as jnp
from jax import lax
from jax.experimental import pallas as pl
from jax.experimental.pallas import tpu as pltpu
from jax.experimental.pallas import tpu_sc as plsc

F = 32
FH = F // 2
NSUB = 16
NW = 32
NBUF = 4
SEC = 64
DEG_GRP = 5
_K_DEG = {51200: 512, 12800: 512, 3200: 320, 800: 160, 200: 40, 17000: 425,
          68200: 440}
_K_SCAT = {102400: 400, 25600: 512, 6400: 256, 1600: 320, 400: 100,
           34000: 425}


def _pad_n(n):
    return ((n + 127) // 128) * 128


@functools.cache
def _deg_kernel(n_pad, e):
    per_w = e // NW
    K = _K_DEG[per_w]
    nch = per_w // K
    rz = n_pad // NSUB
    mesh = plsc.VectorSubcoreMesh(core_axis_name="c", subcore_axis_name="s")

    K_pad = ((K + 15) // 16) * 16

    def body(dstr, zer, out, dst_v, ones_v, sem, acc):
        c = lax.axis_index("c")
        s = lax.axis_index("s")
        w = c * NSUB + s
        pltpu.sync_copy(zer, acc.at[pl.ds(s * rz, rz)])
        pltpu.sync_copy(dstr.at[w], dst_v)
        for i in range(K_pad // 16):
            ones_v[pl.ds(i * 16, 16)] = jnp.ones((16,), jnp.float32)
        ones_s = ones_v if K == K_pad else ones_v.at[pl.ds(0, K)]
        plsc.subcore_barrier()

        def grp(g, carry):
            base = g * DEG_GRP
            for t in range(DEG_GRP):
                pltpu.async_copy(ones_s, acc.at[dst_v.at[base + t]], sem,
                                 add=True)
            for _ in range(DEG_GRP):
                pltpu.make_async_copy(ones_s, acc.at[dst_v.at[0]], sem).wait()
            return carry

        lax.fori_loop(0, nch // DEG_GRP, grp, 0)
        plsc.subcore_barrier()
        pltpu.sync_copy(acc.at[pl.ds(s * rz, rz)],
                        out.at[pl.ds(c * n_pad + s * rz, rz)])

    return pl.kernel(
        body,
        out_type=jax.ShapeDtypeStruct((2 * n_pad,), jnp.float32),
        mesh=mesh,
        compiler_params=pltpu.CompilerParams(use_tc_tiling_on_sc=False),
        scratch_types=[
            pltpu.VMEM((nch, K), jnp.int32),
            pltpu.VMEM((K_pad,), jnp.float32),
            pltpu.SemaphoreType.DMA,
            pltpu.VMEM_SHARED((n_pad,), jnp.float32),
        ],
    )


@functools.cache
def _scat_kernel(n_pad, e):
    per_s = e // NSUB
    K = _K_SCAT[per_s]
    nch = per_s // K
    sec = next(d for d in range(min(nch, SEC), 0, -1) if nch % d == 0)
    nsec = nch // sec
    assert sec >= NBUF
    rz = n_pad // NSUB
    mesh = plsc.VectorSubcoreMesh(core_axis_name="c", subcore_axis_name="s")

    def body(hps, srcr, dstr, zer, out, src_v, dst_v, bufs, gsem, acc):
        c = lax.axis_index("c")
        s = lax.axis_index("s")
        hp_half = hps.at[c]
        pltpu.sync_copy(zer, acc.at[pl.ds(s * rz, rz)])
        plsc.subcore_barrier()
        for t in range(nsec):
            pltpu.sync_copy(srcr.at[s].at[pl.ds(t * sec, sec)], src_v)
            pltpu.sync_copy(dstr.at[s].at[pl.ds(t * sec, sec)], dst_v)
            for b in range(NBUF):
                pltpu.async_copy(hp_half.at[src_v.at[b]], bufs.at[b],
                                 gsem.at[b])

            def step(i, carry):
                p = lax.rem(i, NBUF)
                pltpu.make_async_copy(hp_half.at[src_v.at[0]], bufs.at[p],
                                      gsem.at[p]).wait()
                pltpu.sync_copy(bufs.at[p], acc.at[dst_v.at[i]], add=True)
                nxt = i + NBUF

                @pl.when(nxt < sec)
                def _():
                    pltpu.async_copy(hp_half.at[src_v.at[nxt]], bufs.at[p],
                                     gsem.at[p])

                return carry

            lax.fori_loop(0, sec, step, 0)
        plsc.subcore_barrier()
        pltpu.sync_copy(acc.at[pl.ds(s * rz, rz)],
                        out.at[c].at[pl.ds(s * rz, rz)])

    return pl.kernel(
        body,
        out_type=jax.ShapeDtypeStruct((2, n_pad, FH), jnp.float32),
        mesh=mesh,
        compiler_params=pltpu.CompilerParams(use_tc_tiling_on_sc=False),
        scratch_types=[
            pltpu.VMEM((sec, K), jnp.int32),
            pltpu.VMEM((sec, K), jnp.int32),
            pltpu.VMEM((NBUF, K, FH), jnp.float32),
            pltpu.SemaphoreType.DMA((NBUF,)),
            pltpu.VMEM_SHARED((n_pad, FH), jnp.float32),
        ],
    )


def _tc_block(n_pad):
    b = 6400 if n_pad % 6400 == 0 else n_pad
    return b, n_pad // b


def _split(h):
    return jnp.stack([h[:, :FH], h[:, FH:]], axis=0)


@functools.cache
def _prep_kernel(n_pad, din, interpret=False):
    B, g = _tc_block(n_pad)

    def body(x, d0, d1, w1, dinv_o, hps_o):
        deg = d0[...] + d1[...] + 1.0
        dinv = lax.rsqrt(deg)
        h = jnp.dot(x[...], w1[...], preferred_element_type=jnp.float32)
        dinv_o[...] = dinv
        hps_o[...] = _split(h * dinv)

    return pl.pallas_call(
        body,
        grid=(g,),
        in_specs=[
            pl.BlockSpec((B, din), lambda i: (i, 0)),
            pl.BlockSpec((B, 1), lambda i: (i, 0)),
            pl.BlockSpec((B, 1), lambda i: (i, 0)),
            pl.BlockSpec((din, F), lambda i: (0, 0)),
        ],
        out_specs=[
            pl.BlockSpec((B, 1), lambda i: (i, 0)),
            pl.BlockSpec((2, B, FH), lambda i: (0, i, 0)),
        ],
        out_shape=[
            jax.ShapeDtypeStruct((n_pad, 1), jnp.float32),
            jax.ShapeDtypeStruct((2, n_pad, FH), jnp.float32),
        ],
        interpret=interpret,
    )


@functools.cache
def _mid_kernel(n_pad, interpret=False):
    B, g = _tc_block(n_pad)

    def body(parts, hps, dinv, bvec, w, hpn_o):
        pr = parts[...]
        hr = hps[...]
        agg = jnp.concatenate([pr[0], pr[1]], axis=1)
        hp = jnp.concatenate([hr[0], hr[1]], axis=1)
        z = dinv[...] * (agg + hp) + bvec[...]
        xn = jnp.maximum(z, 0.0)
        hpn = jnp.dot(xn, w[...], preferred_element_type=jnp.float32) * dinv[...]
        hpn_o[...] = _split(hpn)

    return pl.pallas_call(
        body,
        grid=(g,),
        in_specs=[
            pl.BlockSpec((2, B, FH), lambda i: (0, i, 0)),
            pl.BlockSpec((2, B, FH), lambda i: (0, i, 0)),
            pl.BlockSpec((B, 1), lambda i: (i, 0)),
            pl.BlockSpec((1, F), lambda i: (0, 0)),
            pl.BlockSpec((F, F), lambda i: (0, 0)),
        ],
        out_specs=pl.BlockSpec((2, B, FH), lambda i: (0, i, 0)),
        out_shape=jax.ShapeDtypeStruct((2, n_pad, FH), jnp.float32),
        interpret=interpret,
    )


@functools.cache
def _fin_kernel(n_pad, interpret=False):
    B, g = _tc_block(n_pad)

    def body(parts, hps, dinv, bvec, h_o):
        pr = parts[...]
        hr = hps[...]
        agg = jnp.concatenate([pr[0], pr[1]], axis=1)
        hp = jnp.concatenate([hr[0], hr[1]], axis=1)
        h_o[...] = dinv[...] * (agg + hp) + bvec[...]

    return pl.pallas_call(
        body,
        grid=(g,),
        in_specs=[
            pl.BlockSpec((2, B, FH), lambda i: (0, i, 0)),
            pl.BlockSpec((2, B, FH), lambda i: (0, i, 0)),
            pl.BlockSpec((B, 1), lambda i: (i, 0)),
            pl.BlockSpec((1, F), lambda i: (0, 0)),
        ],
        out_specs=pl.BlockSpec((B, F), lambda i: (i, 0)),
        out_shape=jax.ShapeDtypeStruct((n_pad, F), jnp.float32),
        interpret=interpret,
    )


@functools.cache
def _fuse_kernel(n1, interpret=False):
    B = 2048
    g = n1 // B

    def rep(x, k):
        r = x.shape[0]
        return jnp.broadcast_to(x[:, None, :], (r, k, F)).reshape(r * k, F)

    def body(h1, h2, h3, h4, h5, fw, fb, o):
        sm = (h1[...] + rep(h2[...], 4) + rep(h3[...], 16)
              + rep(h4[...], 64) + rep(h5[...], 256))
        sm = jnp.maximum(sm, 0.0)
        o[...] = jnp.dot(sm, fw[...],
                         preferred_element_type=jnp.float32) + fb[...]

    return pl.pallas_call(
        body,
        grid=(g,),
        in_specs=[
            pl.BlockSpec((B, F), lambda i: (i, 0)),
            pl.BlockSpec((B // 4, F), lambda i: (i, 0)),
            pl.BlockSpec((B // 16, F), lambda i: (i, 0)),
            pl.BlockSpec((B // 64, F), lambda i: (i, 0)),
            pl.BlockSpec((B // 256, F), lambda i: (i, 0)),
            pl.BlockSpec((F, 1), lambda i: (0, 0)),
            pl.BlockSpec((1, 1), lambda i: (0, 0)),
        ],
        out_specs=pl.BlockSpec((B, 1), lambda i: (i, 0)),
        out_shape=jax.ShapeDtypeStruct((n1, 1), jnp.float32),
        interpret=interpret,
    )


@jax.jit
def _forward(x1, x2, x3, x4, x5, e1, e2, e3, e4, e5, params):
    xs = (x1, x2, x3, x4, x5)
    es = (e1, e2, e3, e4, e5)
    nl = len(xs)
    Ws_all = [tuple(params['conv%d%d_W' % (i, j)] for j in (1, 2, 3))
              for i in range(1, 6)]
    bs_all = [tuple(params['conv%d%d_b' % (i, j)] for j in (1, 2, 3))
              for i in range(1, 6)]
    n_pads = [_pad_n(x.shape[0]) for x in xs]
    eNs = [e.shape[1] for e in es]

    degps = []
    for e, npad, eN in zip(es, n_pads, eNs):
        kd = _K_DEG[eN // NW]
        dst_d = e[1].reshape(NW, (eN // NW) // kd, kd)
        degps.append(_deg_kernel(npad, eN)(
            dst_d, jnp.zeros((npad // NSUB,), jnp.float32)))

    dinvs, hps_l = [], []
    for x, npad, Ws, degp in zip(xs, n_pads, Ws_all, degps):
        n = x.shape[0]
        xp = x if npad == n else jnp.pad(x, ((0, npad - n), (0, 0)))
        d0 = degp[:npad].reshape(npad, 1)
        d1 = degp[npad:].reshape(npad, 1)
        dinv, hps = _prep_kernel(npad, x.shape[1])(xp, d0, d1, Ws[0])
        dinvs.append(dinv)
        hps_l.append(hps)

    hs = [None] * nl
    for j in range(3):
        parts_l = []
        for e, npad, eN, hps in zip(es, n_pads, eNs, hps_l):
            ks = _K_SCAT[eN // NSUB]
            src_s = e[0].reshape(NSUB, (eN // NSUB) // ks, ks)
            dst_s = e[1].reshape(NSUB, (eN // NSUB) // ks, ks)
            zerf = jnp.zeros((npad // NSUB, FH), jnp.float32)
            parts_l.append(_scat_kernel(npad, eN)(hps, src_s, dst_s, zerf))
        new_hps = []
        for li, (npad, Ws, bs) in enumerate(zip(n_pads, Ws_all, bs_all)):
            bvec = bs[j].reshape(1, F)
            if j < 2:
                new_hps.append(_mid_kernel(npad)(parts_l[li], hps_l[li],
                                                 dinvs[li], bvec, Ws[j + 1]))
            else:
                hs[li] = _fin_kernel(npad)(parts_l[li], hps_l[li],
                                           dinvs[li], bvec)
        hps_l = new_hps

    n1 = x1.shape[0]
    return _fuse_kernel(n1)(hs[0], hs[1], hs[2], hs[3], hs[4],
                            params['fc_W'].reshape(F, 1),
                            params['fc_b'].reshape(1, 1))


def kernel(x1, x2, x3, x4, x5, edge_index1, edge_index2, edge_index3,
           edge_index4, edge_index5, params):
    return _forward(x1, x2, x3, x4, x5, edge_index1, edge_index2,
                    edge_index3, edge_index4, edge_index5, params)


# speedup vs baseline: 1.0205x; 1.0012x over previous
import functools

import jax
import jax.numpy as jnp
from jax import lax
from jax.experimental import pallas as pl
from jax.experimental.pallas import tpu as pltpu
from jax.experimental.pallas import tpu_sc as plsc

F = 32
FH = F // 2
NSUB = 16
NW = 32
NBUF = 4
SEC = 64
_K_DEG = {51200: 512, 12800: 512, 3200: 320, 800: 160, 200: 40, 17000: 425,
          68200: 440}
_K_SCAT = {102400: 400, 25600: 512, 6400: 256, 1600: 320, 400: 100,
           34000: 425}


def _pad_n(n):
    return ((n + 127) // 128) * 128


@functools.cache
def _deg_kernel(n_pad, e):
    per_w = e // NW
    K = _K_DEG[per_w]
    nch = per_w // K
    grp_n = next(g for g in (20, 10, 5) if nch % g == 0)
    rz = n_pad // NSUB
    mesh = plsc.VectorSubcoreMesh(core_axis_name="c", subcore_axis_name="s")

    K_pad = ((K + 15) // 16) * 16

    def body(dstr, zer, out, dst_v, ones_v, sem, acc):
        c = lax.axis_index("c")
        s = lax.axis_index("s")
        w = c * NSUB + s
        pltpu.sync_copy(zer, acc.at[pl.ds(s * rz, rz)])
        pltpu.sync_copy(dstr.at[w], dst_v)
        for i in range(K_pad // 16):
            ones_v[pl.ds(i * 16, 16)] = jnp.ones((16,), jnp.float32)
        ones_s = ones_v if K == K_pad else ones_v.at[pl.ds(0, K)]
        plsc.subcore_barrier()

        def grp(g, carry):
            base = g * grp_n
            for t in range(grp_n):
                pltpu.async_copy(ones_s, acc.at[dst_v.at[base + t]], sem,
                                 add=True)
            for _ in range(grp_n):
                pltpu.make_async_copy(ones_s, acc.at[dst_v.at[0]], sem).wait()
            return carry

        lax.fori_loop(0, nch // grp_n, grp, 0)
        plsc.subcore_barrier()
        pltpu.sync_copy(acc.at[pl.ds(s * rz, rz)],
                        out.at[pl.ds(c * n_pad + s * rz, rz)])

    return pl.kernel(
        body,
        out_type=jax.ShapeDtypeStruct((2 * n_pad,), jnp.float32),
        mesh=mesh,
        compiler_params=pltpu.CompilerParams(use_tc_tiling_on_sc=False),
        scratch_types=[
            pltpu.VMEM((nch, K), jnp.int32),
            pltpu.VMEM((K_pad,), jnp.float32),
            pltpu.SemaphoreType.DMA,
            pltpu.VMEM_SHARED((n_pad,), jnp.float32),
        ],
    )


@functools.cache
def _scat_kernel(n_pad, e):
    per_s = e // NSUB
    K = _K_SCAT[per_s]
    nch = per_s // K
    sec = next(d for d in range(min(nch, SEC), 0, -1) if nch % d == 0)
    nsec = nch // sec
    assert sec >= NBUF
    rz = n_pad // NSUB
    mesh = plsc.VectorSubcoreMesh(core_axis_name="c", subcore_axis_name="s")

    def body(hps, srcr, dstr, zer, out, src_v, dst_v, bufs, gsem, acc):
        c = lax.axis_index("c")
        s = lax.axis_index("s")
        hp_half = hps.at[c]
        pltpu.sync_copy(zer, acc.at[pl.ds(s * rz, rz)])
        plsc.subcore_barrier()
        for t in range(nsec):
            pltpu.sync_copy(srcr.at[s].at[pl.ds(t * sec, sec)], src_v)
            pltpu.sync_copy(dstr.at[s].at[pl.ds(t * sec, sec)], dst_v)
            for b in range(NBUF):
                pltpu.async_copy(hp_half.at[src_v.at[b]], bufs.at[b],
                                 gsem.at[b])

            def step(i, carry):
                p = lax.rem(i, NBUF)
                pltpu.make_async_copy(hp_half.at[src_v.at[0]], bufs.at[p],
                                      gsem.at[p]).wait()
                pltpu.sync_copy(bufs.at[p], acc.at[dst_v.at[i]], add=True)
                nxt = i + NBUF

                @pl.when(nxt < sec)
                def _():
                    pltpu.async_copy(hp_half.at[src_v.at[nxt]], bufs.at[p],
                                     gsem.at[p])

                return carry

            lax.fori_loop(0, sec, step, 0)
        plsc.subcore_barrier()
        pltpu.sync_copy(acc.at[pl.ds(s * rz, rz)],
                        out.at[c].at[pl.ds(s * rz, rz)])

    return pl.kernel(
        body,
        out_type=jax.ShapeDtypeStruct((2, n_pad, FH), jnp.float32),
        mesh=mesh,
        compiler_params=pltpu.CompilerParams(use_tc_tiling_on_sc=False),
        scratch_types=[
            pltpu.VMEM((sec, K), jnp.int32),
            pltpu.VMEM((sec, K), jnp.int32),
            pltpu.VMEM((NBUF, K, FH), jnp.float32),
            pltpu.SemaphoreType.DMA((NBUF,)),
            pltpu.VMEM_SHARED((n_pad, FH), jnp.float32),
        ],
    )


def _tc_block(n_pad):
    b = 6400 if n_pad % 6400 == 0 else n_pad
    return b, n_pad // b


def _split(h):
    return jnp.stack([h[:, :FH], h[:, FH:]], axis=0)


@functools.cache
def _prep_kernel(n_pad, din, interpret=False):
    B, g = _tc_block(n_pad)

    def body(x, d0, d1, w1, dinv_o, hps_o):
        deg = d0[...] + d1[...] + 1.0
        dinv = lax.rsqrt(deg)
        h = jnp.dot(x[...], w1[...], preferred_element_type=jnp.float32)
        dinv_o[...] = dinv
        hps_o[...] = _split(h * dinv)

    return pl.pallas_call(
        body,
        grid=(g,),
        in_specs=[
            pl.BlockSpec((B, din), lambda i: (i, 0)),
            pl.BlockSpec((B, 1), lambda i: (i, 0)),
            pl.BlockSpec((B, 1), lambda i: (i, 0)),
            pl.BlockSpec((din, F), lambda i: (0, 0)),
        ],
        out_specs=[
            pl.BlockSpec((B, 1), lambda i: (i, 0)),
            pl.BlockSpec((2, B, FH), lambda i: (0, i, 0)),
        ],
        out_shape=[
            jax.ShapeDtypeStruct((n_pad, 1), jnp.float32),
            jax.ShapeDtypeStruct((2, n_pad, FH), jnp.float32),
        ],
        interpret=interpret,
    )


@functools.cache
def _mid_kernel(n_pad, interpret=False):
    B, g = _tc_block(n_pad)

    def body(parts, hps, dinv, bvec, w, hpn_o):
        pr = parts[...]
        hr = hps[...]
        agg = jnp.concatenate([pr[0], pr[1]], axis=1)
        hp = jnp.concatenate([hr[0], hr[1]], axis=1)
        z = dinv[...] * (agg + hp) + bvec[...]
        xn = jnp.maximum(z, 0.0)
        hpn = jnp.dot(xn, w[...], preferred_element_type=jnp.float32) * dinv[...]
        hpn_o[...] = _split(hpn)

    return pl.pallas_call(
        body,
        grid=(g,),
        in_specs=[
            pl.BlockSpec((2, B, FH), lambda i: (0, i, 0)),
            pl.BlockSpec((2, B, FH), lambda i: (0, i, 0)),
            pl.BlockSpec((B, 1), lambda i: (i, 0)),
            pl.BlockSpec((1, F), lambda i: (0, 0)),
            pl.BlockSpec((F, F), lambda i: (0, 0)),
        ],
        out_specs=pl.BlockSpec((2, B, FH), lambda i: (0, i, 0)),
        out_shape=jax.ShapeDtypeStruct((2, n_pad, FH), jnp.float32),
        interpret=interpret,
    )


@functools.cache
def _fin_kernel(n_pad, interpret=False):
    B, g = _tc_block(n_pad)

    def body(parts, hps, dinv, bvec, h_o):
        pr = parts[...]
        hr = hps[...]
        agg = jnp.concatenate([pr[0], pr[1]], axis=1)
        hp = jnp.concatenate([hr[0], hr[1]], axis=1)
        h_o[...] = dinv[...] * (agg + hp) + bvec[...]

    return pl.pallas_call(
        body,
        grid=(g,),
        in_specs=[
            pl.BlockSpec((2, B, FH), lambda i: (0, i, 0)),
            pl.BlockSpec((2, B, FH), lambda i: (0, i, 0)),
            pl.BlockSpec((B, 1), lambda i: (i, 0)),
            pl.BlockSpec((1, F), lambda i: (0, 0)),
        ],
        out_specs=pl.BlockSpec((B, F), lambda i: (i, 0)),
        out_shape=jax.ShapeDtypeStruct((n_pad, F), jnp.float32),
        interpret=interpret,
    )


@functools.cache
def _fuse_kernel(n1, interpret=False):
    B = 2048
    g = n1 // B

    def rep(x, k):
        r = x.shape[0]
        return jnp.broadcast_to(x[:, None, :], (r, k, F)).reshape(r * k, F)

    def body(h1, h2, h3, h4, h5, fw, fb, o):
        sm = (h1[...] + rep(h2[...], 4) + rep(h3[...], 16)
              + rep(h4[...], 64) + rep(h5[...], 256))
        sm = jnp.maximum(sm, 0.0)
        o[...] = jnp.dot(sm, fw[...],
                         preferred_element_type=jnp.float32) + fb[...]

    return pl.pallas_call(
        body,
        grid=(g,),
        in_specs=[
            pl.BlockSpec((B, F), lambda i: (i, 0)),
            pl.BlockSpec((B // 4, F), lambda i: (i, 0)),
            pl.BlockSpec((B // 16, F), lambda i: (i, 0)),
            pl.BlockSpec((B // 64, F), lambda i: (i, 0)),
            pl.BlockSpec((B // 256, F), lambda i: (i, 0)),
            pl.BlockSpec((F, 1), lambda i: (0, 0)),
            pl.BlockSpec((1, 1), lambda i: (0, 0)),
        ],
        out_specs=pl.BlockSpec((B, 1), lambda i: (i, 0)),
        out_shape=jax.ShapeDtypeStruct((n1, 1), jnp.float32),
        interpret=interpret,
    )


@jax.jit
def _forward(x1, x2, x3, x4, x5, e1, e2, e3, e4, e5, params):
    xs = (x1, x2, x3, x4, x5)
    es = (e1, e2, e3, e4, e5)
    nl = len(xs)
    Ws_all = [tuple(params['conv%d%d_W' % (i, j)] for j in (1, 2, 3))
              for i in range(1, 6)]
    bs_all = [tuple(params['conv%d%d_b' % (i, j)] for j in (1, 2, 3))
              for i in range(1, 6)]
    n_pads = [_pad_n(x.shape[0]) for x in xs]
    eNs = [e.shape[1] for e in es]

    degps = []
    for e, npad, eN in zip(es, n_pads, eNs):
        kd = _K_DEG[eN // NW]
        dst_d = e[1].reshape(NW, (eN // NW) // kd, kd)
        degps.append(_deg_kernel(npad, eN)(
            dst_d, jnp.zeros((npad // NSUB,), jnp.float32)))

    dinvs, hps_l = [], []
    for x, npad, Ws, degp in zip(xs, n_pads, Ws_all, degps):
        n = x.shape[0]
        xp = x if npad == n else jnp.pad(x, ((0, npad - n), (0, 0)))
        d0 = degp[:npad].reshape(npad, 1)
        d1 = degp[npad:].reshape(npad, 1)
        dinv, hps = _prep_kernel(npad, x.shape[1])(xp, d0, d1, Ws[0])
        dinvs.append(dinv)
        hps_l.append(hps)

    hs = [None] * nl
    for j in range(3):
        parts_l = []
        for e, npad, eN, hps in zip(es, n_pads, eNs, hps_l):
            ks = _K_SCAT[eN // NSUB]
            src_s = e[0].reshape(NSUB, (eN // NSUB) // ks, ks)
            dst_s = e[1].reshape(NSUB, (eN // NSUB) // ks, ks)
            zerf = jnp.zeros((npad // NSUB, FH), jnp.float32)
            parts_l.append(_scat_kernel(npad, eN)(hps, src_s, dst_s, zerf))
        new_hps = []
        for li, (npad, Ws, bs) in enumerate(zip(n_pads, Ws_all, bs_all)):
            bvec = bs[j].reshape(1, F)
            if j < 2:
                new_hps.append(_mid_kernel(npad)(parts_l[li], hps_l[li],
                                                 dinvs[li], bvec, Ws[j + 1]))
            else:
                hs[li] = _fin_kernel(npad)(parts_l[li], hps_l[li],
                                           dinvs[li], bvec)
        hps_l = new_hps

    n1 = x1.shape[0]
    return _fuse_kernel(n1)(hs[0], hs[1], hs[2], hs[3], hs[4],
                            params['fc_W'].reshape(F, 1),
                            params['fc_b'].reshape(1, 1))


def kernel(x1, x2, x3, x4, x5, edge_index1, edge_index2, edge_index3,
           edge_index4, edge_index5, params):
    return _forward(x1, x2, x3, x4, x5, edge_index1, edge_index2,
                    edge_index3, edge_index4, edge_index5, params)
